# Initial kernel scaffold; baseline (speedup 1.0000x reference)
import jax
import jax.numpy as jnp
from jax.experimental import pallas as pl


def kernel(x, experts, gate_w, gate_b):
    raise NotImplementedError("write your pallas kernel here")


# speedup vs baseline: 1.1680x; 1.1680x over previous
import functools

import jax
import jax.numpy as jnp
from jax.experimental import pallas as pl

E = 16
D = 64
K = 4
S_BLK = 128


def _moe_block_kernel(x_ref, w_ref, b_ref, ex_ref, out_ref):
    x = x_ref[0]
    logits = jnp.dot(w_ref[...], x,
                     preferred_element_type=jnp.float32) + b_ref[...]
    m = jnp.max(logits, axis=0, keepdims=True)
    p = jnp.exp(logits - m)
    probs = p / jnp.sum(p, axis=0, keepdims=True)
    pt = probs.T

    iota_e = jax.lax.broadcasted_iota(jnp.int32, pt.shape, 1)
    mask = jnp.zeros(pt.shape, jnp.bool_)
    wp = pt
    for _ in range(K):
        mx = jnp.max(wp, axis=1, keepdims=True)
        sel_idx = jnp.min(jnp.where(wp == mx, iota_e, E), axis=1,
                          keepdims=True)
        sel = iota_e == sel_idx
        mask = jnp.logical_or(mask, sel)
        wp = jnp.where(sel, -jnp.inf, wp)
    pm = jnp.where(mask, pt, 0.0)

    acc = pm[:, 0:1] * ex_ref[0, 0]
    for e in range(1, E):
        acc = acc + pm[:, e:e + 1] * ex_ref[0, e]
    out_ref[0] = acc


@jax.jit
def kernel(x, experts, gate_w, gate_b):
    N, C, H, W = x.shape
    S = H * W
    xs = x.reshape(N, C, S)
    exs = experts.reshape(N, E, S, D)
    b2 = gate_b.reshape(E, 1)
    grid = (N, pl.cdiv(S, S_BLK))
    out = pl.pallas_call(
        _moe_block_kernel,
        grid=grid,
        in_specs=[
            pl.BlockSpec((1, C, S_BLK), lambda n, s: (n, 0, s)),
            pl.BlockSpec((E, C), lambda n, s: (0, 0)),
            pl.BlockSpec((E, 1), lambda n, s: (0, 0)),
            pl.BlockSpec((1, E, S_BLK, D), lambda n, s: (n, 0, s, 0)),
        ],
        out_specs=pl.BlockSpec((1, S_BLK, D), lambda n, s: (n, s, 0)),
        out_shape=jax.ShapeDtypeStruct((N, S, D), jnp.float32),
    )(xs, gate_w, b2, exs)
    return out.reshape(N, H, W, D)
